# baseline (speedup 1.0000x reference)
import functools

import jax
import jax.numpy as jnp
from jax.experimental import pallas as pl
from jax.experimental.pallas import tpu as pltpu

BN = 400
BE = 4000


def _silu(x):
    return x * jax.nn.sigmoid(x)


def _mmT(x, w):
    return jax.lax.dot_general(x, w, (((1,), (1,)), ((), ())),
                               preferred_element_type=jnp.float32)


def _enc_body(nf, w0, b0, w1, b1, ws, wd, h_out, p_out, q_out):
    t = _silu(_mmT(nf[...], w0[...]) + b0[...])
    h = _mmT(t, w1[...]) + b1[...]
    h_out[...] = h
    p_out[...] = _mmT(h, ws[...])
    q_out[...] = _mmT(h, wd[...])


def _enc_call(nf, w0, b0, w1, b1, ws, wd):
    n = nf.shape[0]
    grid = n // BN
    full = lambda r, c: pl.BlockSpec((r, c), lambda i: (0, 0))
    blk = lambda c: pl.BlockSpec((BN, c), lambda i: (i, 0))
    return pl.pallas_call(
        _enc_body,
        grid=(grid,),
        in_specs=[blk(128), full(64, 128), full(1, 64), full(64, 64),
                  full(1, 64), full(64, 64), full(64, 64)],
        out_specs=[blk(64), blk(64), blk(64)],
        out_shape=[jax.ShapeDtypeStruct((n, 64), jnp.float32)] * 3,
    )(nf, w0, b0, w1, b1, ws, wd)


def _edge_body(ps, qd, xs, xd, ea, wq4, we, b1, w2, b2, wx1, bx1, wx24, bx24,
               msg_out, wd_out):
    diff = xs[...] - xd[...]
    d2 = diff * diff
    sqw = jax.lax.dot_general(d2, wq4[...], (((1,), (0,)), ((), ())),
                              preferred_element_type=jnp.float32)
    t = ps[...] + qd[...] + sqw + _mmT(ea[...], we[...]) + b1[...]
    u = _silu(t)
    msg = _silu(_mmT(u, w2[...]) + b2[...])
    msg_out[...] = msg
    v = _silu(_mmT(msg, wx1[...]) + bx1[...])
    w4 = _mmT(v, wx24[...]) + bx24[...]
    wd_out[...] = diff * w4


def _edge_call(ps, qd, xs, xd, ea, lw):
    e = ps.shape[0]
    grid = e // BE
    full = lambda r, c: pl.BlockSpec((r, c), lambda i: (0, 0))
    blk = lambda c: pl.BlockSpec((BE, c), lambda i: (i, 0))
    return pl.pallas_call(
        _edge_body,
        grid=(grid,),
        in_specs=[blk(64), blk(64), blk(4), blk(4), blk(16),
                  full(4, 64), full(64, 16), full(1, 64), full(64, 64),
                  full(1, 64), full(64, 64), full(1, 64), full(4, 64),
                  full(1, 4)],
        out_specs=[blk(64), blk(4)],
        out_shape=[jax.ShapeDtypeStruct((e, 64), jnp.float32),
                   jax.ShapeDtypeStruct((e, 4), jnp.float32)],
    )(ps, qd, xs, xd, ea, lw["Wq4"], lw["We"], lw["b1"], lw["W2"], lw["b2"],
      lw["Wx1"], lw["bx1"], lw["Wx24"], lw["bx24"])


def _node_body(with_tables, h, agg, cnt, cu, x, wha, whb, bh1, wh2, bh2,
               lng, lnb, ws, wd, *outs):
    c = jnp.maximum(cnt[...], 1.0)
    a = agg[...] / c
    hv = h[...]
    t = _silu(_mmT(hv, wha[...]) + _mmT(a, whb[...]) + bh1[...])
    hh = _mmT(t, wh2[...]) + bh2[...]
    pre = hv + hh
    mu = jnp.mean(pre, axis=-1, keepdims=True)
    d = pre - mu
    var = jnp.mean(d * d, axis=-1, keepdims=True)
    hn = d * jax.lax.rsqrt(var + 1e-5) * lng[...] + lnb[...]
    outs[0][...] = hn
    outs[1][...] = x[...] + cu[...] / c
    if with_tables:
        outs[2][...] = _mmT(hn, ws[...])
        outs[3][...] = _mmT(hn, wd[...])


def _node_call(h, agg, cnt, cu, x, lw, ws_next, wd_next):
    n = h.shape[0]
    grid = n // BN
    with_tables = ws_next is not None
    if not with_tables:
        ws_next = jnp.zeros((64, 64), jnp.float32)
        wd_next = ws_next
    full = lambda r, c: pl.BlockSpec((r, c), lambda i: (0, 0))
    blk = lambda c: pl.BlockSpec((BN, c), lambda i: (i, 0))
    n_out = 4 if with_tables else 2
    out_specs = [blk(64), blk(4)] + ([blk(64), blk(64)] if with_tables else [])
    out_shape = ([jax.ShapeDtypeStruct((n, 64), jnp.float32),
                  jax.ShapeDtypeStruct((n, 4), jnp.float32)] +
                 ([jax.ShapeDtypeStruct((n, 64), jnp.float32)] * 2
                  if with_tables else []))
    res = pl.pallas_call(
        functools.partial(_node_body, with_tables),
        grid=(grid,),
        in_specs=[blk(64), blk(64), blk(1), blk(4), blk(4),
                  full(64, 64), full(64, 64), full(1, 64), full(64, 64),
                  full(1, 64), full(1, 64), full(1, 64), full(64, 64),
                  full(64, 64)],
        out_specs=out_specs,
        out_shape=out_shape,
    )(h, agg, cnt, cu, x, lw["Wha"], lw["Whb"], lw["bh1"], lw["Wh2"],
      lw["bh2"], lw["lng"], lw["lnb"], ws_next, wd_next)
    return (tuple(res) + (None, None))[:4]


def _ro_body(gh, gcnt, w0, b0, w1, b1, w2, b2, out):
    g = gh[...] / jnp.maximum(gcnt[...], 1.0)
    t = _silu(_mmT(g, w0[...]) + b0[...])
    t = _silu(_mmT(t, w1[...]) + b1[...])
    out[...] = _mmT(t, w2[...]) + b2[...]


def _ro_call(gh, gcnt, r0, r1, r2):
    g = gh.shape[0]
    full = lambda r, c: pl.BlockSpec((r, c), lambda i: (0, 0))
    return pl.pallas_call(
        _ro_body,
        grid=(1,),
        in_specs=[full(g, 64), full(g, 1), full(64, 64), full(1, 64),
                  full(32, 64), full(1, 32), full(2, 32), full(1, 2)],
        out_specs=full(g, 2),
        out_shape=jax.ShapeDtypeStruct((g, 2), jnp.float32),
    )(gh, gcnt, r0["W"], r0["b"].reshape(1, -1), r1["W"],
      r1["b"].reshape(1, -1), r2["W"], r2["b"].reshape(1, -1))


def _prep_layer(p):
    w1 = p["e1"]["W"]
    return {
        "Ws": w1[:, :64], "Wd": w1[:, 64:128],
        "Wq4": jnp.tile(w1[:, 128].reshape(1, 64), (4, 1)),
        "We": w1[:, 129:145],
        "b1": p["e1"]["b"].reshape(1, 64),
        "W2": p["e2"]["W"], "b2": p["e2"]["b"].reshape(1, 64),
        "Wx1": p["x1"]["W"], "bx1": p["x1"]["b"].reshape(1, 64),
        "Wx24": jnp.tile(p["x2"]["W"].reshape(1, 64), (4, 1)),
        "bx24": jnp.tile(p["x2"]["b"].reshape(1, 1), (1, 4)),
        "Wha": p["h1"]["W"][:, :64], "Whb": p["h1"]["W"][:, 64:],
        "bh1": p["h1"]["b"].reshape(1, 64),
        "Wh2": p["h2"]["W"], "bh2": p["h2"]["b"].reshape(1, 64),
        "lng": p["ln_g"].reshape(1, 64), "lnb": p["ln_b"].reshape(1, 64),
    }


def kernel(node_feats, coords, edge_index, edge_feats, batch, params):
    n = node_feats.shape[0]
    e = edge_index.shape[1]
    n_graphs = 64
    src, dst = edge_index[0], edge_index[1]
    lws = [_prep_layer(p) for p in params["layers"]]

    x = jnp.pad(coords, ((0, 0), (0, 1)))

    cnt = jax.ops.segment_sum(jnp.ones((e,), jnp.float32), dst,
                              num_segments=n).reshape(n, 1)

    h, P, Q = _enc_call(node_feats, params["enc"][0]["W"],
                        params["enc"][0]["b"].reshape(1, 64),
                        params["enc"][1]["W"],
                        params["enc"][1]["b"].reshape(1, 64),
                        lws[0]["Ws"], lws[0]["Wd"])

    for li, lw in enumerate(lws):
        ps = jnp.take(P, src, axis=0)
        qd = jnp.take(Q, dst, axis=0)
        xs = jnp.take(x, src, axis=0)
        xd = jnp.take(x, dst, axis=0)
        msg, wd = _edge_call(ps, qd, xs, xd, edge_feats, lw)
        agg = jax.ops.segment_sum(msg, dst, num_segments=n)
        cu = jax.ops.segment_sum(wd, src, num_segments=n)
        last = li == len(lws) - 1
        ws_next = None if last else lws[li + 1]["Ws"]
        wd_next = None if last else lws[li + 1]["Wd"]
        h, x, P, Q = _node_call(h, agg, cnt, cu, x, lw, ws_next, wd_next)

    gh = jax.ops.segment_sum(h, batch, num_segments=n_graphs)
    gcnt = jax.ops.segment_sum(jnp.ones((n,), jnp.float32), batch,
                               num_segments=n_graphs).reshape(n_graphs, 1)
    r = params["ro"]
    return _ro_call(gh, gcnt, r[0], r[1], r[2])


# speedup vs baseline: 3.5555x; 3.5555x over previous
import functools

import jax
import jax.numpy as jnp
from jax import lax
from jax.experimental import pallas as pl
from jax.experimental.pallas import tpu as pltpu
from jax.experimental.pallas import tpu_sc as plsc

BN = 400
BE = 4000
GC = 128
NW = 32
SPLIT = 25600
TROW = SPLIT
ZROWS = 1600


def _silu(x):
    return x * jax.nn.sigmoid(x)


def _mmT(x, w):
    return lax.dot_general(x, w, (((1,), (1,)), ((), ())),
                           preferred_element_type=jnp.float32)


def _mm(x, w):
    return lax.dot_general(x, w, (((1,), (0,)), ((), ())),
                           preferred_element_type=jnp.float32)


def _sc_mesh():
    return plsc.VectorSubcoreMesh(core_axis_name="c", subcore_axis_name="s",
                                  num_cores=2, num_subcores=16)


def _sc_gather(ts, td, src, dst):
    e = src.shape[0]
    eper = e // NW
    nfull = eper // GC
    tail_off = eper - GC
    nch = nfull + (1 if eper % GC else 0)

    @functools.partial(
        pl.kernel, mesh=_sc_mesh(),
        compiler_params=pltpu.CompilerParams(use_tc_tiling_on_sc=False),
        out_type=[jax.ShapeDtypeStruct((e, 80), jnp.float32),
                  jax.ShapeDtypeStruct((e, 80), jnp.float32)],
        scratch_types=[pltpu.VMEM((GC,), jnp.int32),
                       pltpu.VMEM((GC,), jnp.int32),
                       pltpu.VMEM((GC, 80), jnp.float32),
                       pltpu.VMEM((GC, 80), jnp.float32),
                       pltpu.SemaphoreType.DMA, pltpu.SemaphoreType.DMA],
    )
    def k(ts_h, td_h, src_h, dst_h, os_h, od_h, isv, idv, rs, rd, s1, s2):
        c = lax.axis_index("c")
        s = lax.axis_index("s")
        base = (c * 16 + s) * eper

        def body(j, carry):
            off = base + jnp.minimum(j * GC, tail_off)
            pltpu.sync_copy(src_h.at[pl.ds(off, GC)], isv)
            pltpu.sync_copy(dst_h.at[pl.ds(off, GC)], idv)
            d1 = pltpu.async_copy(ts_h.at[isv], rs, s1)
            d2 = pltpu.async_copy(td_h.at[idv], rd, s2)
            d1.wait()
            d2.wait()
            pltpu.sync_copy(rs, os_h.at[pl.ds(off, GC)])
            pltpu.sync_copy(rd, od_h.at[pl.ds(off, GC)])
            return carry

        lax.fori_loop(0, nch, body, 0)

    return k(ts, td, src, dst)


def _localize(idx_ref, csplit, thr, q):
    pos = lax.iota(jnp.int32, 16) + q * 16
    v = idx_ref[pl.ds(q * 16, 16)] - csplit
    ok = (v >= 0) & (v < SPLIT) & (pos >= thr)
    idx_ref[pl.ds(q * 16, 16)] = jnp.where(ok, v, TROW)


def _sc_segsum(data, idx, z64):
    e, w = data.shape
    eper = e // NW
    nfull = eper // GC
    tail_off = eper - GC
    nch = nfull + (1 if eper % GC else 0)
    overlap = nfull * GC - tail_off

    @functools.partial(
        pl.kernel, mesh=_sc_mesh(),
        compiler_params=pltpu.CompilerParams(use_tc_tiling_on_sc=False),
        out_type=jax.ShapeDtypeStruct((2 * SPLIT, w), jnp.float32),
        scratch_types=[pltpu.VMEM_SHARED((SPLIT + 1, w), jnp.float32),
                       pltpu.VMEM((GC,), jnp.int32),
                       pltpu.VMEM((GC, w), jnp.float32),
                       pltpu.SemaphoreType.DMA, pltpu.SemaphoreType.DMA],
    )
    def k(data_h, idx_h, z_h, out_h, tab_sh, iv, rv, s1, s2):
        c = lax.axis_index("c")
        s = lax.axis_index("s")
        base = (c * 16 + s) * eper
        csplit = c * SPLIT
        tbase = s * ZROWS

        pltpu.sync_copy(z_h.at[:, pl.ds(0, w)], tab_sh.at[pl.ds(tbase, ZROWS)])
        plsc.subcore_barrier()

        def body(j, carry):
            off = base + jnp.minimum(j * GC, tail_off)
            thr = jnp.where(j == nch - 1, overlap, 0)
            pltpu.sync_copy(idx_h.at[pl.ds(off, GC)], iv)
            d1 = pltpu.async_copy(data_h.at[pl.ds(off, GC)], rv, s1)
            for q in range(GC // 16):
                _localize(iv, csplit, thr, q)
            d1.wait()
            d2 = pltpu.async_copy(rv, tab_sh.at[iv], s2, add=True)
            d2.wait()
            return carry

        lax.fori_loop(0, nch, body, 0)
        plsc.subcore_barrier()
        pltpu.sync_copy(tab_sh.at[pl.ds(tbase, ZROWS)],
                        out_h.at[pl.ds(csplit + tbase, ZROWS)])

    return k(data, idx, z64)


def _sc_count(dst, ones, z64):
    e = dst.shape[0]
    eper = e // NW
    nfull = eper // GC
    tail_off = eper - GC
    nch = nfull + (1 if eper % GC else 0)
    overlap = nfull * GC - tail_off

    @functools.partial(
        pl.kernel, mesh=_sc_mesh(),
        compiler_params=pltpu.CompilerParams(use_tc_tiling_on_sc=False),
        out_type=jax.ShapeDtypeStruct((2 * SPLIT, 16), jnp.float32),
        scratch_types=[pltpu.VMEM_SHARED((SPLIT + 1, 16), jnp.float32),
                       pltpu.VMEM((GC,), jnp.int32),
                       pltpu.VMEM((GC, 16), jnp.float32),
                       pltpu.SemaphoreType.DMA],
    )
    def k(dst_h, ones_h, z_h, cnt_h, cnt_sh, dv, ov, s1):
        c = lax.axis_index("c")
        s = lax.axis_index("s")
        base = (c * 16 + s) * eper
        csplit = c * SPLIT
        tbase = s * ZROWS

        pltpu.sync_copy(ones_h, ov)
        pltpu.sync_copy(z_h.at[:, pl.ds(0, 16)], cnt_sh.at[pl.ds(tbase, ZROWS)])
        plsc.subcore_barrier()

        def body(j, carry):
            off = base + jnp.minimum(j * GC, tail_off)
            thr = jnp.where(j == nch - 1, overlap, 0)
            pltpu.sync_copy(dst_h.at[pl.ds(off, GC)], dv)
            for q in range(GC // 16):
                _localize(dv, csplit, thr, q)
            d1 = pltpu.async_copy(ov, cnt_sh.at[dv], s1, add=True)
            d1.wait()
            return carry

        lax.fori_loop(0, nch, body, 0)
        plsc.subcore_barrier()
        pltpu.sync_copy(cnt_sh.at[pl.ds(tbase, ZROWS)],
                        cnt_h.at[pl.ds(csplit + tbase, ZROWS)])

    return k(dst, ones, z64)


def _enc_body(nf, x4, w0, b0, w1, b1, ws, wd, h_out, ts_out, td_out):
    t = _silu(_mmT(nf[...], w0[...]) + b0[...])
    h = _mmT(t, w1[...]) + b1[...]
    h_out[...] = h
    xz = jnp.concatenate([x4[...], jnp.zeros((x4.shape[0], 12), jnp.float32)],
                         axis=1)
    ts_out[...] = jnp.concatenate([_mmT(h, ws[...]), xz], axis=1)
    td_out[...] = jnp.concatenate([_mmT(h, wd[...]), xz], axis=1)


def _enc_call(nf, x4, w0, b0, w1, b1, ws, wd):
    n = nf.shape[0]
    grid = n // BN
    full = lambda r, c: pl.BlockSpec((r, c), lambda i: (0, 0))
    blk = lambda c: pl.BlockSpec((BN, c), lambda i: (i, 0))
    return pl.pallas_call(
        _enc_body,
        grid=(grid,),
        in_specs=[blk(128), blk(4), full(64, 128), full(1, 64), full(64, 64),
                  full(1, 64), full(64, 64), full(64, 64)],
        out_specs=[blk(64), blk(80), blk(80)],
        out_shape=[jax.ShapeDtypeStruct((n, 64), jnp.float32),
                   jax.ShapeDtypeStruct((n, 80), jnp.float32),
                   jax.ShapeDtypeStruct((n, 80), jnp.float32)],
    )(nf, x4, w0, b0, w1, b1, ws, wd)


def _edge_body(trs, trd, ea, wq4, we, b1, w2, b2, wx1, bx1, wx24, bx24,
               msg_out, wd_out):
    ts = trs[...]
    td = trd[...]
    diff = ts[:, 64:68] - td[:, 64:68]
    d2 = diff * diff
    sqw = _mm(d2, wq4[...])
    t = ts[:, 0:64] + td[:, 0:64] + sqw + _mmT(ea[...], we[...]) + b1[...]
    u = _silu(t)
    msg = _silu(_mmT(u, w2[...]) + b2[...])
    msg_out[...] = msg
    v = _silu(_mmT(msg, wx1[...]) + bx1[...])
    w4 = _mmT(v, wx24[...]) + bx24[...]
    wd_out[...] = jnp.concatenate(
        [diff * w4, jnp.zeros((diff.shape[0], 12), jnp.float32)], axis=1)


def _edge_call(rows_s, rows_d, ea, lw):
    e = rows_s.shape[0]
    grid = e // BE
    full = lambda r, c: pl.BlockSpec((r, c), lambda i: (0, 0))
    blk = lambda c: pl.BlockSpec((BE, c), lambda i: (i, 0))
    return pl.pallas_call(
        _edge_body,
        grid=(grid,),
        in_specs=[blk(80), blk(80), blk(16),
                  full(4, 64), full(64, 16), full(1, 64), full(64, 64),
                  full(1, 64), full(64, 64), full(1, 64), full(4, 64),
                  full(1, 4)],
        out_specs=[blk(64), blk(16)],
        out_shape=[jax.ShapeDtypeStruct((e, 64), jnp.float32),
                   jax.ShapeDtypeStruct((e, 16), jnp.float32)],
    )(rows_s, rows_d, ea, lw["Wq4"], lw["We"], lw["b1"], lw["W2"], lw["b2"],
      lw["Wx1"], lw["bx1"], lw["Wx24"], lw["bx24"])


def _node_body(with_tables, h, agg, cnt, cu, x, wha, whb, bh1, wh2, bh2,
               lng, lnb, ws, wd, *outs):
    c = jnp.maximum(cnt[:, 0:1], 1.0)
    a = agg[...] / c
    hv = h[...]
    t = _silu(_mmT(hv, wha[...]) + _mmT(a, whb[...]) + bh1[...])
    hh = _mmT(t, wh2[...]) + bh2[...]
    pre = hv + hh
    mu = jnp.mean(pre, axis=-1, keepdims=True)
    d = pre - mu
    var = jnp.mean(d * d, axis=-1, keepdims=True)
    hn = d * lax.rsqrt(var + 1e-5) * lng[...] + lnb[...]
    outs[0][...] = hn
    xn = x[...] + cu[:, 0:4] / c
    outs[1][...] = xn
    if with_tables:
        xz = jnp.concatenate([xn, jnp.zeros((xn.shape[0], 12), jnp.float32)],
                             axis=1)
        outs[2][...] = jnp.concatenate([_mmT(hn, ws[...]), xz], axis=1)
        outs[3][...] = jnp.concatenate([_mmT(hn, wd[...]), xz], axis=1)


def _node_call(h, agg, cnt, cu, x, lw, ws_next, wd_next):
    n = h.shape[0]
    grid = n // BN
    with_tables = ws_next is not None
    if not with_tables:
        ws_next = jnp.zeros((64, 64), jnp.float32)
        wd_next = ws_next
    full = lambda r, c: pl.BlockSpec((r, c), lambda i: (0, 0))
    blk = lambda c: pl.BlockSpec((BN, c), lambda i: (i, 0))
    out_specs = [blk(64), blk(4)] + ([blk(80), blk(80)] if with_tables else [])
    out_shape = ([jax.ShapeDtypeStruct((n, 64), jnp.float32),
                  jax.ShapeDtypeStruct((n, 4), jnp.float32)] +
                 ([jax.ShapeDtypeStruct((n, 80), jnp.float32)] * 2
                  if with_tables else []))
    res = pl.pallas_call(
        functools.partial(_node_body, with_tables),
        grid=(grid,),
        in_specs=[blk(64), blk(64), blk(16), blk(16), blk(4),
                  full(64, 64), full(64, 64), full(1, 64), full(64, 64),
                  full(1, 64), full(1, 64), full(1, 64), full(64, 64),
                  full(64, 64)],
        out_specs=out_specs,
        out_shape=out_shape,
    )(h, agg, cnt, cu, x, lw["Wha"], lw["Whb"], lw["bh1"], lw["Wh2"],
      lw["bh2"], lw["lng"], lw["lnb"], ws_next, wd_next)
    return (tuple(res) + (None, None))[:4]


def _pool_body(h, b3, gh_out, gc_out):
    i = pl.program_id(0)

    @pl.when(i == 0)
    def _():
        gh_out[...] = jnp.zeros_like(gh_out)
        gc_out[...] = jnp.zeros_like(gc_out)

    hv = h[...]
    bv = b3[...].reshape(1, BN)
    gids = lax.broadcasted_iota(jnp.int32, (64, BN), 0)
    onehot_t = (gids == bv).astype(jnp.float32)
    gh_out[...] += _mm(onehot_t, hv)
    gc_out[...] += _mm(onehot_t, jnp.ones((BN, 64), jnp.float32))


def _pool_call(h, batch3):
    n = h.shape[0]
    grid = n // BN
    return pl.pallas_call(
        _pool_body,
        grid=(grid,),
        in_specs=[pl.BlockSpec((BN, 64), lambda i: (i, 0)),
                  pl.BlockSpec((1, 1, BN), lambda i: (i, 0, 0))],
        out_specs=[pl.BlockSpec((64, 64), lambda i: (0, 0)),
                   pl.BlockSpec((64, 64), lambda i: (0, 0))],
        out_shape=[jax.ShapeDtypeStruct((64, 64), jnp.float32),
                   jax.ShapeDtypeStruct((64, 64), jnp.float32)],
    )(h, batch3)


def _ro_body(gh, gc, w0, b0, w1, b1, w2, b2, out):
    g = gh[...] / jnp.maximum(gc[...], 1.0)
    t = _silu(_mmT(g, w0[...]) + b0[...])
    t = _silu(_mmT(t, w1[...]) + b1[...])
    out[...] = _mmT(t, w2[...]) + b2[...]


def _ro_call(gh, gc, r0, r1, r2):
    g = gh.shape[0]
    full = lambda r, c: pl.BlockSpec((r, c), lambda i: (0, 0))
    return pl.pallas_call(
        _ro_body,
        grid=(1,),
        in_specs=[full(g, 64), full(g, 64), full(64, 64), full(1, 64),
                  full(32, 64), full(1, 32), full(2, 32), full(1, 2)],
        out_specs=full(g, 2),
        out_shape=jax.ShapeDtypeStruct((g, 2), jnp.float32),
    )(gh, gc, r0["W"], r0["b"].reshape(1, -1), r1["W"],
      r1["b"].reshape(1, -1), r2["W"], r2["b"].reshape(1, -1))


def _prep_layer(p):
    w1 = p["e1"]["W"]
    return {
        "Ws": w1[:, :64], "Wd": w1[:, 64:128],
        "Wq4": jnp.tile(w1[:, 128].reshape(1, 64), (4, 1)),
        "We": w1[:, 129:145],
        "b1": p["e1"]["b"].reshape(1, 64),
        "W2": p["e2"]["W"], "b2": p["e2"]["b"].reshape(1, 64),
        "Wx1": p["x1"]["W"], "bx1": p["x1"]["b"].reshape(1, 64),
        "Wx24": jnp.tile(p["x2"]["W"].reshape(1, 64), (4, 1)),
        "bx24": jnp.tile(p["x2"]["b"].reshape(1, 1), (1, 4)),
        "Wha": p["h1"]["W"][:, :64], "Whb": p["h1"]["W"][:, 64:],
        "bh1": p["h1"]["b"].reshape(1, 64),
        "Wh2": p["h2"]["W"], "bh2": p["h2"]["b"].reshape(1, 64),
        "lng": p["ln_g"].reshape(1, 64), "lnb": p["ln_b"].reshape(1, 64),
    }


def kernel(node_feats, coords, edge_index, edge_feats, batch, params):
    n = node_feats.shape[0]
    src = edge_index[0].astype(jnp.int32)
    dst = edge_index[1].astype(jnp.int32)
    lws = [_prep_layer(p) for p in params["layers"]]

    x4 = jnp.pad(coords, ((0, 0), (0, 1)))
    z64 = jnp.zeros((ZROWS, 64), jnp.float32)
    ones16 = jnp.ones((GC, 16), jnp.float32)

    cnt = _sc_count(dst, ones16, z64)

    h, ts, td = _enc_call(node_feats, x4, params["enc"][0]["W"],
                          params["enc"][0]["b"].reshape(1, 64),
                          params["enc"][1]["W"],
                          params["enc"][1]["b"].reshape(1, 64),
                          lws[0]["Ws"], lws[0]["Wd"])
    x = x4
    for li, lw in enumerate(lws):
        rows_s, rows_d = _sc_gather(ts, td, src, dst)
        msg, wd16 = _edge_call(rows_s, rows_d, edge_feats, lw)
        agg = _sc_segsum(msg, dst, z64)
        cu = _sc_segsum(wd16, src, z64)
        last = li == len(lws) - 1
        ws_next = None if last else lws[li + 1]["Ws"]
        wd_next = None if last else lws[li + 1]["Wd"]
        h, x, ts, td = _node_call(h, agg, cnt, cu, x, lw, ws_next, wd_next)

    batch3 = batch.astype(jnp.int32).reshape(n // BN, 1, BN)
    gh, gc = _pool_call(h, batch3)
    r = params["ro"]
    return _ro_call(gh, gc, r[0], r[1], r[2])


# speedup vs baseline: 3.9169x; 1.1016x over previous
import functools

import jax
import jax.numpy as jnp
from jax import lax
from jax.experimental import pallas as pl
from jax.experimental.pallas import tpu as pltpu
from jax.experimental.pallas import tpu_sc as plsc

BN = 400
BE = 4000
GC = 128
NSUB = 4
NW = 32
SPLIT = 25600
TROW = SPLIT
ZROWS = 1600


def _silu(x):
    return x * jax.nn.sigmoid(x)


def _mmT(x, w):
    return lax.dot_general(x, w, (((1,), (1,)), ((), ())),
                           preferred_element_type=jnp.float32)


def _mm(x, w):
    return lax.dot_general(x, w, (((1,), (0,)), ((), ())),
                           preferred_element_type=jnp.float32)


def _sc_mesh():
    return plsc.VectorSubcoreMesh(core_axis_name="c", subcore_axis_name="s",
                                  num_cores=2, num_subcores=16)


def _sc_gather(ts, td, src, dst):
    e = src.shape[0]
    eper = e // NW
    big = GC * NSUB
    nfull = eper // big
    tail_off = eper - big
    nch = nfull + (1 if eper % big else 0)

    @functools.partial(
        pl.kernel, mesh=_sc_mesh(),
        compiler_params=pltpu.CompilerParams(use_tc_tiling_on_sc=False),
        out_type=[jax.ShapeDtypeStruct((e, 80), jnp.float32),
                  jax.ShapeDtypeStruct((e, 80), jnp.float32)],
        scratch_types=[[pltpu.VMEM((GC,), jnp.int32)] * NSUB,
                       [pltpu.VMEM((GC,), jnp.int32)] * NSUB,
                       pltpu.VMEM((big, 80), jnp.float32),
                       pltpu.VMEM((big, 80), jnp.float32),
                       pltpu.SemaphoreType.DMA, pltpu.SemaphoreType.DMA],
    )
    def k(ts_h, td_h, src_h, dst_h, os_h, od_h, isv, idv, rs, rd, s1, s2):
        c = lax.axis_index("c")
        s = lax.axis_index("s")
        base = (c * 16 + s) * eper

        def body(j, carry):
            off = base + jnp.minimum(j * big, tail_off)
            ds_ = []
            for q in range(NSUB):
                pltpu.sync_copy(src_h.at[pl.ds(off + q * GC, GC)], isv[q])
                pltpu.sync_copy(dst_h.at[pl.ds(off + q * GC, GC)], idv[q])
                ds_.append(pltpu.async_copy(
                    ts_h.at[isv[q]], rs.at[pl.ds(q * GC, GC)], s1))
                ds_.append(pltpu.async_copy(
                    td_h.at[idv[q]], rd.at[pl.ds(q * GC, GC)], s2))
            for d in ds_:
                d.wait()
            pltpu.sync_copy(rs, os_h.at[pl.ds(off, big)])
            pltpu.sync_copy(rd, od_h.at[pl.ds(off, big)])
            return carry

        lax.fori_loop(0, nch, body, 0)

    return k(ts, td, src, dst)


def _localize(idx_ref, csplit, thr, posbase, q):
    pos = lax.iota(jnp.int32, 16) + (posbase + q * 16)
    v = idx_ref[pl.ds(q * 16, 16)] - csplit
    ok = (v >= 0) & (v < SPLIT) & (pos >= thr)
    idx_ref[pl.ds(q * 16, 16)] = jnp.where(ok, v, TROW)


def _sc_segsum(data, idx, z64, nsub):
    e, w = data.shape
    eper = e // NW
    big = GC * nsub
    nfull = eper // big
    tail_off = eper - big
    nch = nfull + (1 if eper % big else 0)
    overlap = nfull * big - tail_off

    @functools.partial(
        pl.kernel, mesh=_sc_mesh(),
        compiler_params=pltpu.CompilerParams(use_tc_tiling_on_sc=False),
        out_type=jax.ShapeDtypeStruct((2 * SPLIT, w), jnp.float32),
        scratch_types=[pltpu.VMEM_SHARED((SPLIT + 1, w), jnp.float32),
                       [pltpu.VMEM((GC,), jnp.int32)] * nsub,
                       pltpu.VMEM((big, w), jnp.float32),
                       pltpu.SemaphoreType.DMA, pltpu.SemaphoreType.DMA],
    )
    def k(data_h, idx_h, z_h, out_h, tab_sh, iv, rv, s1, s2):
        c = lax.axis_index("c")
        s = lax.axis_index("s")
        base = (c * 16 + s) * eper
        csplit = c * SPLIT
        tbase = s * ZROWS

        pltpu.sync_copy(z_h.at[:, pl.ds(0, w)], tab_sh.at[pl.ds(tbase, ZROWS)])
        plsc.subcore_barrier()

        def body(j, carry):
            off = base + jnp.minimum(j * big, tail_off)
            thr = jnp.where(j == nch - 1, overlap, 0)
            d1 = pltpu.async_copy(data_h.at[pl.ds(off, big)], rv, s1)
            for q in range(nsub):
                pltpu.sync_copy(idx_h.at[pl.ds(off + q * GC, GC)], iv[q])
                for p in range(GC // 16):
                    _localize(iv[q], csplit, thr, q * GC, p)
            d1.wait()
            ds_ = [pltpu.async_copy(rv.at[pl.ds(q * GC, GC)],
                                    tab_sh.at[iv[q]], s2, add=True)
                   for q in range(nsub)]
            for d in ds_:
                d.wait()
            return carry

        lax.fori_loop(0, nch, body, 0)
        plsc.subcore_barrier()
        pltpu.sync_copy(tab_sh.at[pl.ds(tbase, ZROWS)],
                        out_h.at[pl.ds(csplit + tbase, ZROWS)])

    return k(data, idx, z64)


def _sc_count(dst, ones, z64):
    e = dst.shape[0]
    eper = e // NW
    nsub = 8
    big = GC * nsub
    nfull = eper // big
    tail_off = eper - big
    nch = nfull + (1 if eper % big else 0)
    overlap = nfull * big - tail_off

    @functools.partial(
        pl.kernel, mesh=_sc_mesh(),
        compiler_params=pltpu.CompilerParams(use_tc_tiling_on_sc=False),
        out_type=jax.ShapeDtypeStruct((2 * SPLIT, 16), jnp.float32),
        scratch_types=[pltpu.VMEM_SHARED((SPLIT + 1, 16), jnp.float32),
                       [pltpu.VMEM((GC,), jnp.int32)] * nsub,
                       pltpu.VMEM((GC, 16), jnp.float32),
                       pltpu.SemaphoreType.DMA],
    )
    def k(dst_h, ones_h, z_h, cnt_h, cnt_sh, dv, ov, s1):
        c = lax.axis_index("c")
        s = lax.axis_index("s")
        base = (c * 16 + s) * eper
        csplit = c * SPLIT
        tbase = s * ZROWS

        pltpu.sync_copy(ones_h, ov)
        pltpu.sync_copy(z_h.at[:, pl.ds(0, 16)], cnt_sh.at[pl.ds(tbase, ZROWS)])
        plsc.subcore_barrier()

        def body(j, carry):
            off = base + jnp.minimum(j * big, tail_off)
            thr = jnp.where(j == nch - 1, overlap, 0)
            ds_ = []
            for q in range(nsub):
                pltpu.sync_copy(dst_h.at[pl.ds(off + q * GC, GC)], dv[q])
                for p in range(GC // 16):
                    _localize(dv[q], csplit, thr, q * GC, p)
                ds_.append(pltpu.async_copy(ov, cnt_sh.at[dv[q]], s1,
                                            add=True))
            for d in ds_:
                d.wait()
            return carry

        lax.fori_loop(0, nch, body, 0)
        plsc.subcore_barrier()
        pltpu.sync_copy(cnt_sh.at[pl.ds(tbase, ZROWS)],
                        cnt_h.at[pl.ds(csplit + tbase, ZROWS)])

    return k(dst, ones, z64)


def _enc_body(nf, x4, w0, b0, w1, b1, ws, wd, h_out, ts_out, td_out):
    t = _silu(_mmT(nf[...], w0[...]) + b0[...])
    h = _mmT(t, w1[...]) + b1[...]
    h_out[...] = h
    xz = jnp.concatenate([x4[...], jnp.zeros((x4.shape[0], 12), jnp.float32)],
                         axis=1)
    ts_out[...] = jnp.concatenate([_mmT(h, ws[...]), xz], axis=1)
    td_out[...] = jnp.concatenate([_mmT(h, wd[...]), xz], axis=1)


def _enc_call(nf, x4, w0, b0, w1, b1, ws, wd):
    n = nf.shape[0]
    grid = n // BN
    full = lambda r, c: pl.BlockSpec((r, c), lambda i: (0, 0))
    blk = lambda c: pl.BlockSpec((BN, c), lambda i: (i, 0))
    return pl.pallas_call(
        _enc_body,
        grid=(grid,),
        in_specs=[blk(128), blk(4), full(64, 128), full(1, 64), full(64, 64),
                  full(1, 64), full(64, 64), full(64, 64)],
        out_specs=[blk(64), blk(80), blk(80)],
        out_shape=[jax.ShapeDtypeStruct((n, 64), jnp.float32),
                   jax.ShapeDtypeStruct((n, 80), jnp.float32),
                   jax.ShapeDtypeStruct((n, 80), jnp.float32)],
    )(nf, x4, w0, b0, w1, b1, ws, wd)


def _edge_body(trs, trd, ea, wq4, we, b1, w2, b2, wx1, bx1, wx24, bx24,
               msg_out, wd_out):
    ts = trs[...]
    td = trd[...]
    diff = ts[:, 64:68] - td[:, 64:68]
    d2 = diff * diff
    sqw = _mm(d2, wq4[...])
    t = ts[:, 0:64] + td[:, 0:64] + sqw + _mmT(ea[...], we[...]) + b1[...]
    u = _silu(t)
    msg = _silu(_mmT(u, w2[...]) + b2[...])
    msg_out[...] = msg
    v = _silu(_mmT(msg, wx1[...]) + bx1[...])
    w4 = _mmT(v, wx24[...]) + bx24[...]
    wd_out[...] = jnp.concatenate(
        [diff * w4, jnp.zeros((diff.shape[0], 12), jnp.float32)], axis=1)


def _edge_call(rows_s, rows_d, ea, lw):
    e = rows_s.shape[0]
    grid = e // BE
    full = lambda r, c: pl.BlockSpec((r, c), lambda i: (0, 0))
    blk = lambda c: pl.BlockSpec((BE, c), lambda i: (i, 0))
    return pl.pallas_call(
        _edge_body,
        grid=(grid,),
        in_specs=[blk(80), blk(80), blk(16),
                  full(4, 64), full(64, 16), full(1, 64), full(64, 64),
                  full(1, 64), full(64, 64), full(1, 64), full(4, 64),
                  full(1, 4)],
        out_specs=[blk(64), blk(16)],
        out_shape=[jax.ShapeDtypeStruct((e, 64), jnp.float32),
                   jax.ShapeDtypeStruct((e, 16), jnp.float32)],
    )(rows_s, rows_d, ea, lw["Wq4"], lw["We"], lw["b1"], lw["W2"], lw["b2"],
      lw["Wx1"], lw["bx1"], lw["Wx24"], lw["bx24"])


def _node_body(with_tables, h, agg, cnt, cu, x, wha, whb, bh1, wh2, bh2,
               lng, lnb, ws, wd, *outs):
    c = jnp.maximum(cnt[:, 0:1], 1.0)
    a = agg[...] / c
    hv = h[...]
    t = _silu(_mmT(hv, wha[...]) + _mmT(a, whb[...]) + bh1[...])
    hh = _mmT(t, wh2[...]) + bh2[...]
    pre = hv + hh
    mu = jnp.mean(pre, axis=-1, keepdims=True)
    d = pre - mu
    var = jnp.mean(d * d, axis=-1, keepdims=True)
    hn = d * lax.rsqrt(var + 1e-5) * lng[...] + lnb[...]
    outs[0][...] = hn
    xn = x[...] + cu[:, 0:4] / c
    outs[1][...] = xn
    if with_tables:
        xz = jnp.concatenate([xn, jnp.zeros((xn.shape[0], 12), jnp.float32)],
                             axis=1)
        outs[2][...] = jnp.concatenate([_mmT(hn, ws[...]), xz], axis=1)
        outs[3][...] = jnp.concatenate([_mmT(hn, wd[...]), xz], axis=1)


def _node_call(h, agg, cnt, cu, x, lw, ws_next, wd_next):
    n = h.shape[0]
    grid = n // BN
    with_tables = ws_next is not None
    if not with_tables:
        ws_next = jnp.zeros((64, 64), jnp.float32)
        wd_next = ws_next
    full = lambda r, c: pl.BlockSpec((r, c), lambda i: (0, 0))
    blk = lambda c: pl.BlockSpec((BN, c), lambda i: (i, 0))
    out_specs = [blk(64), blk(4)] + ([blk(80), blk(80)] if with_tables else [])
    out_shape = ([jax.ShapeDtypeStruct((n, 64), jnp.float32),
                  jax.ShapeDtypeStruct((n, 4), jnp.float32)] +
                 ([jax.ShapeDtypeStruct((n, 80), jnp.float32)] * 2
                  if with_tables else []))
    res = pl.pallas_call(
        functools.partial(_node_body, with_tables),
        grid=(grid,),
        in_specs=[blk(64), blk(64), blk(16), blk(16), blk(4),
                  full(64, 64), full(64, 64), full(1, 64), full(64, 64),
                  full(1, 64), full(1, 64), full(1, 64), full(64, 64),
                  full(64, 64)],
        out_specs=out_specs,
        out_shape=out_shape,
    )(h, agg, cnt, cu, x, lw["Wha"], lw["Whb"], lw["bh1"], lw["Wh2"],
      lw["bh2"], lw["lng"], lw["lnb"], ws_next, wd_next)
    return (tuple(res) + (None, None))[:4]


def _pool_body(h, b3, gh_out, gc_out):
    i = pl.program_id(0)

    @pl.when(i == 0)
    def _():
        gh_out[...] = jnp.zeros_like(gh_out)
        gc_out[...] = jnp.zeros_like(gc_out)

    hv = h[...]
    bv = b3[...].reshape(1, BN)
    gids = lax.broadcasted_iota(jnp.int32, (64, BN), 0)
    onehot_t = (gids == bv).astype(jnp.float32)
    gh_out[...] += _mm(onehot_t, hv)
    gc_out[...] += _mm(onehot_t, jnp.ones((BN, 64), jnp.float32))


def _pool_call(h, batch3):
    n = h.shape[0]
    grid = n // BN
    return pl.pallas_call(
        _pool_body,
        grid=(grid,),
        in_specs=[pl.BlockSpec((BN, 64), lambda i: (i, 0)),
                  pl.BlockSpec((1, 1, BN), lambda i: (i, 0, 0))],
        out_specs=[pl.BlockSpec((64, 64), lambda i: (0, 0)),
                   pl.BlockSpec((64, 64), lambda i: (0, 0))],
        out_shape=[jax.ShapeDtypeStruct((64, 64), jnp.float32),
                   jax.ShapeDtypeStruct((64, 64), jnp.float32)],
    )(h, batch3)


def _ro_body(gh, gc, w0, b0, w1, b1, w2, b2, out):
    g = gh[...] / jnp.maximum(gc[...], 1.0)
    t = _silu(_mmT(g, w0[...]) + b0[...])
    t = _silu(_mmT(t, w1[...]) + b1[...])
    out[...] = _mmT(t, w2[...]) + b2[...]


def _ro_call(gh, gc, r0, r1, r2):
    g = gh.shape[0]
    full = lambda r, c: pl.BlockSpec((r, c), lambda i: (0, 0))
    return pl.pallas_call(
        _ro_body,
        grid=(1,),
        in_specs=[full(g, 64), full(g, 64), full(64, 64), full(1, 64),
                  full(32, 64), full(1, 32), full(2, 32), full(1, 2)],
        out_specs=full(g, 2),
        out_shape=jax.ShapeDtypeStruct((g, 2), jnp.float32),
    )(gh, gc, r0["W"], r0["b"].reshape(1, -1), r1["W"],
      r1["b"].reshape(1, -1), r2["W"], r2["b"].reshape(1, -1))


def _prep_layer(p):
    w1 = p["e1"]["W"]
    return {
        "Ws": w1[:, :64], "Wd": w1[:, 64:128],
        "Wq4": jnp.tile(w1[:, 128].reshape(1, 64), (4, 1)),
        "We": w1[:, 129:145],
        "b1": p["e1"]["b"].reshape(1, 64),
        "W2": p["e2"]["W"], "b2": p["e2"]["b"].reshape(1, 64),
        "Wx1": p["x1"]["W"], "bx1": p["x1"]["b"].reshape(1, 64),
        "Wx24": jnp.tile(p["x2"]["W"].reshape(1, 64), (4, 1)),
        "bx24": jnp.tile(p["x2"]["b"].reshape(1, 1), (1, 4)),
        "Wha": p["h1"]["W"][:, :64], "Whb": p["h1"]["W"][:, 64:],
        "bh1": p["h1"]["b"].reshape(1, 64),
        "Wh2": p["h2"]["W"], "bh2": p["h2"]["b"].reshape(1, 64),
        "lng": p["ln_g"].reshape(1, 64), "lnb": p["ln_b"].reshape(1, 64),
    }


def kernel(node_feats, coords, edge_index, edge_feats, batch, params):
    n = node_feats.shape[0]
    src = edge_index[0].astype(jnp.int32)
    dst = edge_index[1].astype(jnp.int32)
    lws = [_prep_layer(p) for p in params["layers"]]

    x4 = jnp.pad(coords, ((0, 0), (0, 1)))
    z64 = jnp.zeros((ZROWS, 64), jnp.float32)
    ones16 = jnp.ones((GC, 16), jnp.float32)

    cnt = _sc_count(dst, ones16, z64)

    h, ts, td = _enc_call(node_feats, x4, params["enc"][0]["W"],
                          params["enc"][0]["b"].reshape(1, 64),
                          params["enc"][1]["W"],
                          params["enc"][1]["b"].reshape(1, 64),
                          lws[0]["Ws"], lws[0]["Wd"])
    x = x4
    for li, lw in enumerate(lws):
        rows_s, rows_d = _sc_gather(ts, td, src, dst)
        msg, wd16 = _edge_call(rows_s, rows_d, edge_feats, lw)
        agg = _sc_segsum(msg, dst, z64, 3)
        cu = _sc_segsum(wd16, src, z64, 8)
        last = li == len(lws) - 1
        ws_next = None if last else lws[li + 1]["Ws"]
        wd_next = None if last else lws[li + 1]["Wd"]
        h, x, ts, td = _node_call(h, agg, cnt, cu, x, lw, ws_next, wd_next)

    batch3 = batch.astype(jnp.int32).reshape(n // BN, 1, BN)
    gh, gc = _pool_call(h, batch3)
    r = params["ro"]
    return _ro_call(gh, gc, r[0], r[1], r[2])


# speedup vs baseline: 4.1795x; 1.0671x over previous
import functools

import jax
import jax.numpy as jnp
from jax import lax
from jax.experimental import pallas as pl
from jax.experimental.pallas import tpu as pltpu
from jax.experimental.pallas import tpu_sc as plsc

BN = 400
BE = 3200
EHALF = (409600, 390400)
GC = 128
NSUB = 4
NW = 32
SPLIT = 25600
TROW = SPLIT
ZROWS = 1600


def _silu(x):
    return x * jax.nn.sigmoid(x)


def _mmT(x, w):
    return lax.dot_general(x, w, (((1,), (1,)), ((), ())),
                           preferred_element_type=jnp.float32)


def _mm(x, w):
    return lax.dot_general(x, w, (((1,), (0,)), ((), ())),
                           preferred_element_type=jnp.float32)


def _sc_mesh():
    return plsc.VectorSubcoreMesh(core_axis_name="c", subcore_axis_name="s",
                                  num_cores=2, num_subcores=16)


def _sc_gather(ts, td, src, dst, e0, ne):
    eper = ne // NW
    big = GC * NSUB
    nfull = eper // big
    tail_off = eper - big
    nch = nfull + (1 if eper % big else 0)

    @functools.partial(
        pl.kernel, mesh=_sc_mesh(),
        compiler_params=pltpu.CompilerParams(use_tc_tiling_on_sc=False),
        out_type=[jax.ShapeDtypeStruct((ne, 80), jnp.float32),
                  jax.ShapeDtypeStruct((ne, 80), jnp.float32)],
        scratch_types=[[pltpu.VMEM((GC,), jnp.int32)] * NSUB,
                       [pltpu.VMEM((GC,), jnp.int32)] * NSUB,
                       pltpu.VMEM((big, 80), jnp.float32),
                       pltpu.VMEM((big, 80), jnp.float32),
                       pltpu.SemaphoreType.DMA, pltpu.SemaphoreType.DMA],
    )
    def k(ts_h, td_h, src_h, dst_h, os_h, od_h, isv, idv, rs, rd, s1, s2):
        c = lax.axis_index("c")
        s = lax.axis_index("s")
        base = (c * 16 + s) * eper

        def body(j, carry):
            off = base + jnp.minimum(j * big, tail_off)
            ds_ = []
            for q in range(NSUB):
                pltpu.sync_copy(src_h.at[pl.ds(e0 + off + q * GC, GC)], isv[q])
                pltpu.sync_copy(dst_h.at[pl.ds(e0 + off + q * GC, GC)], idv[q])
                ds_.append(pltpu.async_copy(
                    ts_h.at[isv[q]], rs.at[pl.ds(q * GC, GC)], s1))
                ds_.append(pltpu.async_copy(
                    td_h.at[idv[q]], rd.at[pl.ds(q * GC, GC)], s2))
            for d in ds_:
                d.wait()
            pltpu.sync_copy(rs, os_h.at[pl.ds(off, big)])
            pltpu.sync_copy(rd, od_h.at[pl.ds(off, big)])
            return carry

        lax.fori_loop(0, nch, body, 0)

    return k(ts, td, src, dst)


def _localize(idx_ref, csplit, thr, posbase, q):
    pos = lax.iota(jnp.int32, 16) + (posbase + q * 16)
    v = idx_ref[pl.ds(q * 16, 16)] - csplit
    ok = (v >= 0) & (v < SPLIT) & (pos >= thr)
    idx_ref[pl.ds(q * 16, 16)] = jnp.where(ok, v, TROW)


def _sc_segsum(data, idx, z64, nsub, e0=0):
    e, w = data.shape
    eper = e // NW
    big = GC * nsub
    nfull = eper // big
    tail_off = eper - big
    nch = nfull + (1 if eper % big else 0)
    overlap = nfull * big - tail_off

    @functools.partial(
        pl.kernel, mesh=_sc_mesh(),
        compiler_params=pltpu.CompilerParams(use_tc_tiling_on_sc=False),
        out_type=jax.ShapeDtypeStruct((2 * SPLIT, w), jnp.float32),
        scratch_types=[pltpu.VMEM_SHARED((SPLIT + 1, w), jnp.float32),
                       [pltpu.VMEM((GC,), jnp.int32)] * nsub,
                       pltpu.VMEM((big, w), jnp.float32),
                       pltpu.SemaphoreType.DMA, pltpu.SemaphoreType.DMA],
    )
    def k(data_h, idx_h, z_h, out_h, tab_sh, iv, rv, s1, s2):
        c = lax.axis_index("c")
        s = lax.axis_index("s")
        base = (c * 16 + s) * eper
        csplit = c * SPLIT
        tbase = s * ZROWS

        pltpu.sync_copy(z_h.at[:, pl.ds(0, w)], tab_sh.at[pl.ds(tbase, ZROWS)])
        plsc.subcore_barrier()

        def body(j, carry):
            off = base + jnp.minimum(j * big, tail_off)
            thr = jnp.where(j == nch - 1, overlap, 0)
            d1 = pltpu.async_copy(data_h.at[pl.ds(off, big)], rv, s1)
            for q in range(nsub):
                pltpu.sync_copy(idx_h.at[pl.ds(e0 + off + q * GC, GC)], iv[q])
                for p in range(GC // 16):
                    _localize(iv[q], csplit, thr, q * GC, p)
            d1.wait()
            ds_ = [pltpu.async_copy(rv.at[pl.ds(q * GC, GC)],
                                    tab_sh.at[iv[q]], s2, add=True)
                   for q in range(nsub)]
            for d in ds_:
                d.wait()
            return carry

        lax.fori_loop(0, nch, body, 0)
        plsc.subcore_barrier()
        pltpu.sync_copy(tab_sh.at[pl.ds(tbase, ZROWS)],
                        out_h.at[pl.ds(csplit + tbase, ZROWS)])

    return k(data, idx, z64)


def _sc_count(dst, ones, z64):
    e = dst.shape[0]
    eper = e // NW
    nsub = 8
    big = GC * nsub
    nfull = eper // big
    tail_off = eper - big
    nch = nfull + (1 if eper % big else 0)
    overlap = nfull * big - tail_off

    @functools.partial(
        pl.kernel, mesh=_sc_mesh(),
        compiler_params=pltpu.CompilerParams(use_tc_tiling_on_sc=False),
        out_type=jax.ShapeDtypeStruct((2 * SPLIT, 16), jnp.float32),
        scratch_types=[pltpu.VMEM_SHARED((SPLIT + 1, 16), jnp.float32),
                       [pltpu.VMEM((GC,), jnp.int32)] * nsub,
                       pltpu.VMEM((GC, 16), jnp.float32),
                       pltpu.SemaphoreType.DMA],
    )
    def k(dst_h, ones_h, z_h, cnt_h, cnt_sh, dv, ov, s1):
        c = lax.axis_index("c")
        s = lax.axis_index("s")
        base = (c * 16 + s) * eper
        csplit = c * SPLIT
        tbase = s * ZROWS

        pltpu.sync_copy(ones_h, ov)
        pltpu.sync_copy(z_h.at[:, pl.ds(0, 16)], cnt_sh.at[pl.ds(tbase, ZROWS)])
        plsc.subcore_barrier()

        def body(j, carry):
            off = base + jnp.minimum(j * big, tail_off)
            thr = jnp.where(j == nch - 1, overlap, 0)
            ds_ = []
            for q in range(nsub):
                pltpu.sync_copy(dst_h.at[pl.ds(off + q * GC, GC)], dv[q])
                for p in range(GC // 16):
                    _localize(dv[q], csplit, thr, q * GC, p)
                ds_.append(pltpu.async_copy(ov, cnt_sh.at[dv[q]], s1,
                                            add=True))
            for d in ds_:
                d.wait()
            return carry

        lax.fori_loop(0, nch, body, 0)
        plsc.subcore_barrier()
        pltpu.sync_copy(cnt_sh.at[pl.ds(tbase, ZROWS)],
                        cnt_h.at[pl.ds(csplit + tbase, ZROWS)])

    return k(dst, ones, z64)


def _enc_body(nf, x4, w0, b0, w1, b1, ws, wd, h_out, ts_out, td_out):
    t = _silu(_mmT(nf[...], w0[...]) + b0[...])
    h = _mmT(t, w1[...]) + b1[...]
    h_out[...] = h
    xz = jnp.concatenate([x4[...], jnp.zeros((x4.shape[0], 12), jnp.float32)],
                         axis=1)
    ts_out[...] = jnp.concatenate([_mmT(h, ws[...]), xz], axis=1)
    td_out[...] = jnp.concatenate([_mmT(h, wd[...]), xz], axis=1)


def _enc_call(nf, x4, w0, b0, w1, b1, ws, wd):
    n = nf.shape[0]
    grid = n // BN
    full = lambda r, c: pl.BlockSpec((r, c), lambda i: (0, 0))
    blk = lambda c: pl.BlockSpec((BN, c), lambda i: (i, 0))
    return pl.pallas_call(
        _enc_body,
        grid=(grid,),
        in_specs=[blk(128), blk(4), full(64, 128), full(1, 64), full(64, 64),
                  full(1, 64), full(64, 64), full(64, 64)],
        out_specs=[blk(64), blk(80), blk(80)],
        out_shape=[jax.ShapeDtypeStruct((n, 64), jnp.float32),
                   jax.ShapeDtypeStruct((n, 80), jnp.float32),
                   jax.ShapeDtypeStruct((n, 80), jnp.float32)],
    )(nf, x4, w0, b0, w1, b1, ws, wd)


def _edge_body(trs, trd, ea, wq4, we, b1, w2, b2, wx1, bx1, wx24, bx24,
               msg_out, wd_out):
    ts = trs[...]
    td = trd[...]
    diff = ts[:, 64:68] - td[:, 64:68]
    d2 = diff * diff
    sqw = _mm(d2, wq4[...])
    t = ts[:, 0:64] + td[:, 0:64] + sqw + _mmT(ea[...], we[...]) + b1[...]
    u = _silu(t)
    msg = _silu(_mmT(u, w2[...]) + b2[...])
    msg_out[...] = msg
    v = _silu(_mmT(msg, wx1[...]) + bx1[...])
    w4 = _mmT(v, wx24[...]) + bx24[...]
    wd_out[...] = jnp.concatenate(
        [diff * w4, jnp.zeros((diff.shape[0], 12), jnp.float32)], axis=1)


def _edge_call(rows_s, rows_d, ea, lw):
    e = rows_s.shape[0]
    grid = e // BE
    full = lambda r, c: pl.BlockSpec((r, c), lambda i: (0, 0))
    blk = lambda c: pl.BlockSpec((BE, c), lambda i: (i, 0))
    return pl.pallas_call(
        _edge_body,
        grid=(grid,),
        in_specs=[blk(80), blk(80), blk(16),
                  full(4, 64), full(64, 16), full(1, 64), full(64, 64),
                  full(1, 64), full(64, 64), full(1, 64), full(4, 64),
                  full(1, 4)],
        out_specs=[blk(64), blk(16)],
        out_shape=[jax.ShapeDtypeStruct((e, 64), jnp.float32),
                   jax.ShapeDtypeStruct((e, 16), jnp.float32)],
    )(rows_s, rows_d, ea, lw["Wq4"], lw["We"], lw["b1"], lw["W2"], lw["b2"],
      lw["Wx1"], lw["bx1"], lw["Wx24"], lw["bx24"])


def _node_body(with_tables, h, agg, agg2, cnt, cu, cu2, x, wha, whb, bh1,
               wh2, bh2, lng, lnb, ws, wd, *outs):
    c = jnp.maximum(cnt[:, 0:1], 1.0)
    a = (agg[...] + agg2[...]) / c
    hv = h[...]
    t = _silu(_mmT(hv, wha[...]) + _mmT(a, whb[...]) + bh1[...])
    hh = _mmT(t, wh2[...]) + bh2[...]
    pre = hv + hh
    mu = jnp.mean(pre, axis=-1, keepdims=True)
    d = pre - mu
    var = jnp.mean(d * d, axis=-1, keepdims=True)
    hn = d * lax.rsqrt(var + 1e-5) * lng[...] + lnb[...]
    outs[0][...] = hn
    xn = x[...] + (cu[:, 0:4] + cu2[:, 0:4]) / c
    outs[1][...] = xn
    if with_tables:
        xz = jnp.concatenate([xn, jnp.zeros((xn.shape[0], 12), jnp.float32)],
                             axis=1)
        outs[2][...] = jnp.concatenate([_mmT(hn, ws[...]), xz], axis=1)
        outs[3][...] = jnp.concatenate([_mmT(hn, wd[...]), xz], axis=1)


def _node_call(h, agg, agg2, cnt, cu, cu2, x, lw, ws_next, wd_next):
    n = h.shape[0]
    grid = n // BN
    with_tables = ws_next is not None
    if not with_tables:
        ws_next = jnp.zeros((64, 64), jnp.float32)
        wd_next = ws_next
    full = lambda r, c: pl.BlockSpec((r, c), lambda i: (0, 0))
    blk = lambda c: pl.BlockSpec((BN, c), lambda i: (i, 0))
    out_specs = [blk(64), blk(4)] + ([blk(80), blk(80)] if with_tables else [])
    out_shape = ([jax.ShapeDtypeStruct((n, 64), jnp.float32),
                  jax.ShapeDtypeStruct((n, 4), jnp.float32)] +
                 ([jax.ShapeDtypeStruct((n, 80), jnp.float32)] * 2
                  if with_tables else []))
    res = pl.pallas_call(
        functools.partial(_node_body, with_tables),
        grid=(grid,),
        in_specs=[blk(64), blk(64), blk(64), blk(16), blk(16), blk(16),
                  blk(4),
                  full(64, 64), full(64, 64), full(1, 64), full(64, 64),
                  full(1, 64), full(1, 64), full(1, 64), full(64, 64),
                  full(64, 64)],
        out_specs=out_specs,
        out_shape=out_shape,
    )(h, agg, agg2, cnt, cu, cu2, x, lw["Wha"], lw["Whb"], lw["bh1"], lw["Wh2"],
      lw["bh2"], lw["lng"], lw["lnb"], ws_next, wd_next)
    return (tuple(res) + (None, None))[:4]


def _pool_body(h, b3, gh_out, gc_out):
    i = pl.program_id(0)

    @pl.when(i == 0)
    def _():
        gh_out[...] = jnp.zeros_like(gh_out)
        gc_out[...] = jnp.zeros_like(gc_out)

    hv = h[...]
    bv = b3[...].reshape(1, BN)
    gids = lax.broadcasted_iota(jnp.int32, (64, BN), 0)
    onehot_t = (gids == bv).astype(jnp.float32)
    gh_out[...] += _mm(onehot_t, hv)
    gc_out[...] += _mm(onehot_t, jnp.ones((BN, 64), jnp.float32))


def _pool_call(h, batch3):
    n = h.shape[0]
    grid = n // BN
    return pl.pallas_call(
        _pool_body,
        grid=(grid,),
        in_specs=[pl.BlockSpec((BN, 64), lambda i: (i, 0)),
                  pl.BlockSpec((1, 1, BN), lambda i: (i, 0, 0))],
        out_specs=[pl.BlockSpec((64, 64), lambda i: (0, 0)),
                   pl.BlockSpec((64, 64), lambda i: (0, 0))],
        out_shape=[jax.ShapeDtypeStruct((64, 64), jnp.float32),
                   jax.ShapeDtypeStruct((64, 64), jnp.float32)],
    )(h, batch3)


def _ro_body(gh, gc, w0, b0, w1, b1, w2, b2, out):
    g = gh[...] / jnp.maximum(gc[...], 1.0)
    t = _silu(_mmT(g, w0[...]) + b0[...])
    t = _silu(_mmT(t, w1[...]) + b1[...])
    out[...] = _mmT(t, w2[...]) + b2[...]


def _ro_call(gh, gc, r0, r1, r2):
    g = gh.shape[0]
    full = lambda r, c: pl.BlockSpec((r, c), lambda i: (0, 0))
    return pl.pallas_call(
        _ro_body,
        grid=(1,),
        in_specs=[full(g, 64), full(g, 64), full(64, 64), full(1, 64),
                  full(32, 64), full(1, 32), full(2, 32), full(1, 2)],
        out_specs=full(g, 2),
        out_shape=jax.ShapeDtypeStruct((g, 2), jnp.float32),
    )(gh, gc, r0["W"], r0["b"].reshape(1, -1), r1["W"],
      r1["b"].reshape(1, -1), r2["W"], r2["b"].reshape(1, -1))


def _prep_layer(p):
    w1 = p["e1"]["W"]
    return {
        "Ws": w1[:, :64], "Wd": w1[:, 64:128],
        "Wq4": jnp.tile(w1[:, 128].reshape(1, 64), (4, 1)),
        "We": w1[:, 129:145],
        "b1": p["e1"]["b"].reshape(1, 64),
        "W2": p["e2"]["W"], "b2": p["e2"]["b"].reshape(1, 64),
        "Wx1": p["x1"]["W"], "bx1": p["x1"]["b"].reshape(1, 64),
        "Wx24": jnp.tile(p["x2"]["W"].reshape(1, 64), (4, 1)),
        "bx24": jnp.tile(p["x2"]["b"].reshape(1, 1), (1, 4)),
        "Wha": p["h1"]["W"][:, :64], "Whb": p["h1"]["W"][:, 64:],
        "bh1": p["h1"]["b"].reshape(1, 64),
        "Wh2": p["h2"]["W"], "bh2": p["h2"]["b"].reshape(1, 64),
        "lng": p["ln_g"].reshape(1, 64), "lnb": p["ln_b"].reshape(1, 64),
    }


def kernel(node_feats, coords, edge_index, edge_feats, batch, params):
    n = node_feats.shape[0]
    src = edge_index[0].astype(jnp.int32)
    dst = edge_index[1].astype(jnp.int32)
    lws = [_prep_layer(p) for p in params["layers"]]

    x4 = jnp.pad(coords, ((0, 0), (0, 1)))
    z64 = jnp.zeros((ZROWS, 64), jnp.float32)
    ones16 = jnp.ones((GC, 16), jnp.float32)

    cnt = _sc_count(dst, ones16, z64)

    h, ts, td = _enc_call(node_feats, x4, params["enc"][0]["W"],
                          params["enc"][0]["b"].reshape(1, 64),
                          params["enc"][1]["W"],
                          params["enc"][1]["b"].reshape(1, 64),
                          lws[0]["Ws"], lws[0]["Wd"])
    x = x4
    ea_halves = (edge_feats[:EHALF[0]], edge_feats[EHALF[0]:])
    for li, lw in enumerate(lws):
        aggs, cus = [], []
        for hi, (e0, ne) in enumerate(((0, EHALF[0]), (EHALF[0], EHALF[1]))):
            rows_s, rows_d = _sc_gather(ts, td, src, dst, e0, ne)
            msg, wd16 = _edge_call(rows_s, rows_d, ea_halves[hi], lw)
            aggs.append(_sc_segsum(msg, dst, z64, 3, e0))
            cus.append(_sc_segsum(wd16, src, z64, 8, e0))
        last = li == len(lws) - 1
        ws_next = None if last else lws[li + 1]["Ws"]
        wd_next = None if last else lws[li + 1]["Wd"]
        h, x, ts, td = _node_call(h, aggs[0], aggs[1], cnt, cus[0], cus[1],
                                  x, lw, ws_next, wd_next)

    batch3 = batch.astype(jnp.int32).reshape(n // BN, 1, BN)
    gh, gc = _pool_call(h, batch3)
    r = params["ro"]
    return _ro_call(gh, gc, r[0], r[1], r[2])
